# Initial kernel scaffold; baseline (speedup 1.0000x reference)
import functools

import jax
import jax.numpy as jnp
from jax import lax
from jax.experimental import pallas as pl
from jax.experimental.pallas import tpu as pltpu
from jax.experimental.pallas import tpu_sc as plsc

B, N, K = 4, 2048, 32
D_IN, D_OUT = 128, 128
RQ = 256
NB = N // RQ
_BIG = jnp.float32(3.0e38)

_NC, _NS = 2, 16
_NW = _NC * _NS
_TOT = B * N * K
_RPW = _TOT // _NW
_CH = 128
_NCH = _RPW // _CH


def _fold_body(posW2_ref, W1bot_ref, b1_ref, posb2_ref, Wf_ref, cb_ref):
    W1bot = W1bot_ref[...]
    Wf_ref[...] = jnp.dot(posW2_ref[...], W1bot,
                          preferred_element_type=jnp.float32)
    cb_ref[...] = b1_ref[...] + jnp.dot(posb2_ref[...], W1bot,
                                        preferred_element_type=jnp.float32)


def _prep_body(p_ref, pt_ref, f_ref, posW1_ref, W1top_ref, cb_ref,
               qtab_ref, gtab_ref, idx_ref):
    b = pl.program_id(0)
    p = p_ref[0]
    pt = pt_ref[0]
    qtab_ref[0] = jnp.dot(p, posW1_ref[...],
                          preferred_element_type=jnp.float32)
    gtab_ref[0] = jnp.dot(f_ref[0], W1top_ref[...],
                          preferred_element_type=jnp.float32) + cb_ref[...]
    pn = jnp.sum(pt * pt, axis=0, keepdims=True)
    dots = jnp.dot(p, pt, preferred_element_type=jnp.float32)
    vals0 = pn - 2.0 * dots
    lane = lax.broadcasted_iota(jnp.int32, (RQ, N), 1)
    base = b * N

    def tk_body(k, vals):
        m = jnp.min(vals, axis=1, keepdims=True)
        am = jnp.min(jnp.where(vals == m, lane, jnp.int32(N)),
                     axis=1, keepdims=True)
        idx_ref[0, :, pl.ds(k, 1)] = am + base
        return jnp.where(lane == am, _BIG, vals)

    lax.fori_loop(0, K, tk_body, vals0)


def _sc_gather_body(idx_hbm, qtab_hbm, gtab_hbm, outq_hbm, outg_hbm,
                    idx_v, qrows, grows, sem_q, sem_g):
    wid = lax.axis_index("s") * _NC + lax.axis_index("c")
    base = wid * _RPW

    def chunk(c, carry):
        off = pl.multiple_of(base + c * _CH, _CH)
        pltpu.sync_copy(idx_hbm.at[pl.ds(off, _CH)], idx_v)
        cq = pltpu.async_copy(qtab_hbm.at[idx_v], qrows, sem_q)
        cg = pltpu.async_copy(gtab_hbm.at[idx_v], grows, sem_g)
        cq.wait()
        cg.wait()
        pltpu.sync_copy(qrows, outq_hbm.at[pl.ds(off, _CH)])
        pltpu.sync_copy(grows, outg_hbm.at[pl.ds(off, _CH)])
        return carry

    lax.fori_loop(0, _NCH, chunk, 0)


def _agg_body(gq_ref, gg_ref, qtab_ref, Wf_ref, pb1_ref, W2_ref, b2_ref,
              out_ref):
    qc = qtab_ref[0]
    offs = pb1_ref[...] - qc

    def body(k, acc):
        a = jnp.maximum(gq_ref[0, :, pl.ds(k, 1), :].reshape(RQ, 64) + offs,
                        0.0)
        z = gg_ref[0, :, pl.ds(k, 1), :].reshape(RQ, 128) + jnp.dot(
            a, Wf_ref[...], preferred_element_type=jnp.float32)
        return acc + jnp.maximum(z, 0.0)

    acc = lax.fori_loop(0, K, body, jnp.zeros((RQ, D_OUT), jnp.float32))
    out_ref[0] = jnp.dot(acc * (1.0 / K), W2_ref[...],
                         preferred_element_type=jnp.float32) + b2_ref[...]


_fold_call = pl.pallas_call(
    _fold_body,
    out_shape=[jax.ShapeDtypeStruct((64, 128), jnp.float32),
               jax.ShapeDtypeStruct((1, 128), jnp.float32)],
)

_prep_call = pl.pallas_call(
    _prep_body,
    grid=(B, NB),
    in_specs=[
        pl.BlockSpec((1, RQ, 3), lambda b, nb: (b, nb, 0)),
        pl.BlockSpec((1, 3, N), lambda b, nb: (b, 0, 0)),
        pl.BlockSpec((1, RQ, D_IN), lambda b, nb: (b, nb, 0)),
        pl.BlockSpec((3, 64), lambda b, nb: (0, 0)),
        pl.BlockSpec((D_IN, 128), lambda b, nb: (0, 0)),
        pl.BlockSpec((1, 128), lambda b, nb: (0, 0)),
    ],
    out_specs=[
        pl.BlockSpec((1, RQ, 64), lambda b, nb: (b, nb, 0)),
        pl.BlockSpec((1, RQ, 128), lambda b, nb: (b, nb, 0)),
        pl.BlockSpec((1, RQ, K), lambda b, nb: (b, nb, 0)),
    ],
    out_shape=[jax.ShapeDtypeStruct((B, N, 64), jnp.float32),
               jax.ShapeDtypeStruct((B, N, 128), jnp.float32),
               jax.ShapeDtypeStruct((B, N, K), jnp.int32)],
)

_agg_call = pl.pallas_call(
    _agg_body,
    grid=(B, NB),
    in_specs=[
        pl.BlockSpec((1, RQ, K, 64), lambda b, nb: (b, nb, 0, 0)),
        pl.BlockSpec((1, RQ, K, 128), lambda b, nb: (b, nb, 0, 0)),
        pl.BlockSpec((1, RQ, 64), lambda b, nb: (b, nb, 0)),
        pl.BlockSpec((64, 128), lambda b, nb: (0, 0)),
        pl.BlockSpec((1, 64), lambda b, nb: (0, 0)),
        pl.BlockSpec((128, 128), lambda b, nb: (0, 0)),
        pl.BlockSpec((1, 128), lambda b, nb: (0, 0)),
    ],
    out_specs=pl.BlockSpec((1, RQ, D_OUT), lambda b, nb: (b, nb, 0)),
    out_shape=jax.ShapeDtypeStruct((B, N, D_OUT), jnp.float32),
)

_sc_gather = functools.partial(
    pl.kernel,
    out_type=[jax.ShapeDtypeStruct((_TOT, 64), jnp.float32),
              jax.ShapeDtypeStruct((_TOT, 128), jnp.float32)],
    mesh=plsc.VectorSubcoreMesh(core_axis_name="c", subcore_axis_name="s"),
    scratch_types=[pltpu.VMEM((_CH,), jnp.int32),
                   pltpu.VMEM((_CH, 64), jnp.float32),
                   pltpu.VMEM((_CH, 128), jnp.float32),
                   pltpu.SemaphoreType.DMA,
                   pltpu.SemaphoreType.DMA],
)(_sc_gather_body)


def kernel(points, features, density, pos_W1, pos_b1, pos_W2, pos_b2,
           mlp_W1, mlp_b1, mlp_W2, mlp_b2,
           dw_W1, dw_b1, dw_W2, dw_b2, dw_W3, dw_b3):
    del density, dw_W1, dw_b1, dw_W2, dw_b2, dw_W3, dw_b3
    pT = points.transpose(0, 2, 1)
    W1_top = mlp_W1[:D_IN]
    W1_bot = mlp_W1[D_IN:]
    Wf, cb = _fold_call(pos_W2, W1_bot, mlp_b1.reshape(1, 128),
                        pos_b2.reshape(1, 64))
    qtab, gtab, idx = _prep_call(points, pT, features, pos_W1, W1_top, cb)
    gq, gg = _sc_gather(idx.reshape(_TOT),
                        qtab.reshape(B * N, 64),
                        gtab.reshape(B * N, 128))
    out = _agg_call(gq.reshape(B, N, K, 64), gg.reshape(B, N, K, 128),
                    qtab, Wf, pos_b1.reshape(1, 64), mlp_W2,
                    mlp_b2.reshape(1, 128))
    return out


# speedup vs baseline: 5.7002x; 5.7002x over previous
import functools

import jax
import jax.numpy as jnp
from jax import lax
from jax.experimental import pallas as pl
from jax.experimental.pallas import tpu as pltpu
from jax.experimental.pallas import tpu_sc as plsc

B, N, K = 4, 2048, 32
D_IN, D_OUT = 128, 128
RQ = 256
NB = N // RQ
TW = 256
_BIG = 3.0e38

_NC, _NS = 2, 16
_NW = _NC * _NS
_TOT = B * N * K
_RPW = _TOT // _NW
_CH = 128
_NCH = _RPW // _CH


def _fold_body(posW2_ref, W1bot_ref, b1_ref, posb2_ref, Wf_ref, cb_ref):
    W1bot = W1bot_ref[...]
    Wf_ref[...] = jnp.dot(posW2_ref[...], W1bot,
                          preferred_element_type=jnp.float32,
                          precision=lax.Precision.HIGHEST)
    cb_ref[...] = b1_ref[...] + jnp.dot(posb2_ref[...], W1bot,
                                        preferred_element_type=jnp.float32,
                          precision=lax.Precision.HIGHEST)


def _prep_body(pq_ref, pc_ref, ptq_ref, f_ref, posW1_ref, W1top_ref, cb_ref,
               ttab_ref, idx_ref, vals_ref):
    b = pl.program_id(0)
    k = pl.program_id(2)

    @pl.when(k == 0)
    def _init():
        ttab_ref[0, :, 0:64] = jnp.dot(pq_ref[0], posW1_ref[...],
                                       preferred_element_type=jnp.float32,
                          precision=lax.Precision.HIGHEST)
        ttab_ref[0, :, 64:128] = jnp.zeros((RQ, 64), jnp.float32)
        ttab_ref[0, :, 128:256] = jnp.dot(
            f_ref[0], W1top_ref[...],
            preferred_element_type=jnp.float32,
                          precision=lax.Precision.HIGHEST) + cb_ref[...]
        pc = pc_ref[0]
        pn = jnp.sum(pc * pc, axis=1, keepdims=True)
        dots = jnp.dot(pc, ptq_ref[0],
                       preferred_element_type=jnp.float32,
                          precision=lax.Precision.HIGHEST)
        vals_ref[...] = pn - 2.0 * dots

    vals = vals_ref[...]
    m = jnp.min(vals, axis=0, keepdims=True)
    sub = lax.broadcasted_iota(jnp.int32, (N, RQ), 0)
    am = jnp.min(jnp.where(vals == m, sub, jnp.int32(N)),
                 axis=0, keepdims=True)
    idx_ref[...] = (am + b * N).reshape(1, 1, 1, RQ)
    vals_ref[...] = jnp.where(sub == am, _BIG, vals)


def _sc_gather_body(idx_hbm, ttab_hbm, out_hbm, idx_v, rows_v, sem):
    wid = lax.axis_index("s") * _NC + lax.axis_index("c")
    base = wid * _RPW

    def chunk(c, carry):
        off = pl.multiple_of(base + c * _CH, _CH)
        pltpu.sync_copy(idx_hbm.at[pl.ds(off, _CH)], idx_v)
        pltpu.async_copy(ttab_hbm.at[idx_v], rows_v, sem).wait()
        pltpu.sync_copy(rows_v, out_hbm.at[pl.ds(off, _CH)])
        return carry

    lax.fori_loop(0, _NCH, chunk, 0)


def _agg_body(gt_ref, ttab_ref, Wf_ref, pb1_ref, W2_ref, b2_ref,
              out_ref, acc_ref):
    k = pl.program_id(2)

    @pl.when(k == 0)
    def _init():
        acc_ref[...] = jnp.zeros((RQ, D_OUT), jnp.float32)

    row = gt_ref[0, 0]
    qc = ttab_ref[0, :, 0:64]
    a = jnp.maximum(row[:, 0:64] - qc + pb1_ref[...], 0.0)
    z = row[:, 128:256] + jnp.dot(a, Wf_ref[...],
                                  preferred_element_type=jnp.float32,
                          precision=lax.Precision.HIGHEST)
    acc_ref[...] += jnp.maximum(z, 0.0)

    @pl.when(k == K - 1)
    def _fin():
        out_ref[0] = jnp.dot(acc_ref[...] * (1.0 / K), W2_ref[...],
                             preferred_element_type=jnp.float32,
                          precision=lax.Precision.HIGHEST) + b2_ref[...]


_fold_call = pl.pallas_call(
    _fold_body,
    out_shape=[jax.ShapeDtypeStruct((64, 128), jnp.float32),
               jax.ShapeDtypeStruct((1, 128), jnp.float32)],
)

_prep_call = pl.pallas_call(
    _prep_body,
    grid=(B, NB, K),
    in_specs=[
        pl.BlockSpec((1, RQ, 3), lambda b, nb, k: (b, nb, 0)),
        pl.BlockSpec((1, N, 3), lambda b, nb, k: (b, 0, 0)),
        pl.BlockSpec((1, 3, RQ), lambda b, nb, k: (b, 0, nb)),
        pl.BlockSpec((1, RQ, D_IN), lambda b, nb, k: (b, nb, 0)),
        pl.BlockSpec((3, 64), lambda b, nb, k: (0, 0)),
        pl.BlockSpec((D_IN, 128), lambda b, nb, k: (0, 0)),
        pl.BlockSpec((1, 128), lambda b, nb, k: (0, 0)),
    ],
    out_specs=[
        pl.BlockSpec((1, RQ, TW), lambda b, nb, k: (b, nb, 0)),
        pl.BlockSpec((1, 1, 1, RQ), lambda b, nb, k: (b, k, 0, nb)),
    ],
    out_shape=[jax.ShapeDtypeStruct((B, N, TW), jnp.float32),
               jax.ShapeDtypeStruct((B, K, 1, N), jnp.int32)],
    scratch_shapes=[pltpu.VMEM((N, RQ), jnp.float32)],
)

_agg_call = pl.pallas_call(
    _agg_body,
    grid=(B, NB, K),
    in_specs=[
        pl.BlockSpec((1, 1, RQ, TW), lambda b, nb, k: (b, k, nb, 0)),
        pl.BlockSpec((1, RQ, TW), lambda b, nb, k: (b, nb, 0)),
        pl.BlockSpec((64, 128), lambda b, nb, k: (0, 0)),
        pl.BlockSpec((1, 64), lambda b, nb, k: (0, 0)),
        pl.BlockSpec((128, 128), lambda b, nb, k: (0, 0)),
        pl.BlockSpec((1, 128), lambda b, nb, k: (0, 0)),
    ],
    out_specs=pl.BlockSpec((1, RQ, D_OUT), lambda b, nb, k: (b, nb, 0)),
    out_shape=jax.ShapeDtypeStruct((B, N, D_OUT), jnp.float32),
    scratch_shapes=[pltpu.VMEM((RQ, D_OUT), jnp.float32)],
)


@functools.cache
def _sc_gather_call():
    return functools.partial(
        pl.kernel,
        out_type=jax.ShapeDtypeStruct((_TOT, TW), jnp.float32),
        mesh=plsc.VectorSubcoreMesh(core_axis_name="c",
                                    subcore_axis_name="s"),
        scratch_types=[pltpu.VMEM((_CH,), jnp.int32),
                       pltpu.VMEM((_CH, TW), jnp.float32),
                       pltpu.SemaphoreType.DMA],
    )(_sc_gather_body)


def kernel(points, features, density, pos_W1, pos_b1, pos_W2, pos_b2,
           mlp_W1, mlp_b1, mlp_W2, mlp_b2,
           dw_W1, dw_b1, dw_W2, dw_b2, dw_W3, dw_b3):
    del density, dw_W1, dw_b1, dw_W2, dw_b2, dw_W3, dw_b3
    pT = points.transpose(0, 2, 1)
    W1_top = mlp_W1[:D_IN]
    W1_bot = mlp_W1[D_IN:]
    Wf, cb = _fold_call(pos_W2, W1_bot, mlp_b1.reshape(1, 128),
                        pos_b2.reshape(1, 64))
    ttab, idx = _prep_call(points, points, pT, features, pos_W1, W1_top, cb)
    gt = _sc_gather_call()(idx.reshape(_TOT), ttab.reshape(B * N, TW))
    out = _agg_call(gt.reshape(B, K, N, TW), ttab, Wf,
                    pos_b1.reshape(1, 64), mlp_W2, mlp_b2.reshape(1, 128))
    return out


# speedup vs baseline: 11.9788x; 2.1015x over previous
import functools

import jax
import jax.numpy as jnp
from jax import lax
from jax.experimental import pallas as pl
from jax.experimental.pallas import tpu as pltpu
from jax.experimental.pallas import tpu_sc as plsc

B, N, K = 4, 2048, 32
D_IN, D_OUT = 128, 128
RQ = 256
NB = N // RQ
KG = 8
AQ = 128
NAB = N // AQ
TW = 256
_BIG = 3.0e38

_NC, _NS = 2, 16
_NW = _NC * _NS
_TOTB = N * K
_RPW = _TOTB // _NW
_CH = 128
_NCH = _RPW // _CH


def _fold_body(posW2_ref, W1bot_ref, b1_ref, posb2_ref, Wf_ref, cb_ref):
    W1bot = W1bot_ref[...]
    Wf_ref[...] = jnp.dot(posW2_ref[...], W1bot,
                          preferred_element_type=jnp.float32,
                          precision=lax.Precision.HIGHEST)
    cb_ref[...] = b1_ref[...] + jnp.dot(posb2_ref[...], W1bot,
                                        preferred_element_type=jnp.float32,
                                        precision=lax.Precision.HIGHEST)


def _prep_body(pq_ref, pc_ref, ptq_ref, f_ref, posW1_ref, W1top_ref, cb_ref,
               ttab_ref, idx_ref, vals_ref):
    kg = pl.program_id(1)

    @pl.when(kg == 0)
    def _init():
        ttab_ref[:, 0:64] = jnp.dot(pq_ref[...], posW1_ref[...],
                                    preferred_element_type=jnp.float32,
                                    precision=lax.Precision.HIGHEST)
        ttab_ref[:, 64:128] = jnp.zeros((RQ, 64), jnp.float32)
        ttab_ref[:, 128:256] = jnp.dot(
            f_ref[...], W1top_ref[...],
            preferred_element_type=jnp.float32,
            precision=lax.Precision.HIGHEST) + cb_ref[...]
        pc = pc_ref[...]
        pn = jnp.sum(pc * pc, axis=1, keepdims=True)
        dots = jnp.dot(pc, ptq_ref[...],
                       preferred_element_type=jnp.float32,
                       precision=lax.Precision.HIGHEST)
        vals_ref[...] = pn - 2.0 * dots

    vals = vals_ref[...]
    sub = lax.broadcasted_iota(jnp.int32, (N, RQ), 0)
    for j in range(KG):
        m = jnp.min(vals, axis=0, keepdims=True)
        eq = vals == m
        am = jnp.min(jnp.where(eq, sub, jnp.int32(N)),
                     axis=0, keepdims=True)
        idx_ref[0, j] = am.reshape(RQ)
        vals = jnp.where(eq, _BIG, vals)
    vals_ref[...] = vals


def _sc_gather_body(idx_hbm, ttab_hbm, out_hbm, idx_v, rows_v, sem0, sem1):
    wid = lax.axis_index("s") * _NC + lax.axis_index("c")
    base = wid * _RPW
    sems = (sem0, sem1)

    for bsel in range(2):
        off0 = pl.multiple_of(base + bsel * _CH, _CH)
        pltpu.sync_copy(idx_hbm.at[pl.ds(off0, _CH)], idx_v.at[bsel])
        pltpu.async_copy(ttab_hbm.at[idx_v.at[bsel]], rows_v.at[bsel],
                         sems[bsel])

    def pair(g, carry):
        for bsel in range(2):
            c = g * 2 + bsel
            off = pl.multiple_of(base + c * _CH, _CH)
            pltpu.make_async_copy(ttab_hbm.at[idx_v.at[bsel]],
                                  rows_v.at[bsel], sems[bsel]).wait()
            pltpu.sync_copy(rows_v.at[bsel], out_hbm.at[pl.ds(off, _CH)])

            @pl.when(c + 2 < _NCH)
            def _next():
                offn = pl.multiple_of(base + (c + 2) * _CH, _CH)
                pltpu.sync_copy(idx_hbm.at[pl.ds(offn, _CH)], idx_v.at[bsel])
                pltpu.async_copy(ttab_hbm.at[idx_v.at[bsel]],
                                 rows_v.at[bsel], sems[bsel])
        return carry

    lax.fori_loop(0, _NCH // 2, pair, 0)


def _agg_body(gt_ref, ttab_ref, Wf_ref, pb1_ref, W2_ref, b2_ref, out_ref):
    gt = gt_ref[...]
    qc = ttab_ref[:, 0:64]
    a = jnp.maximum(gt[:, :, 0:64] - qc[None] + pb1_ref[...][None], 0.0)
    mm = jnp.dot(a.reshape(K * AQ, 64), Wf_ref[...],
                 preferred_element_type=jnp.float32,
                 precision=lax.Precision.HIGHEST)
    z = gt[:, :, 128:256] + mm.reshape(K, AQ, 128)
    s = jnp.sum(jnp.maximum(z, 0.0), axis=0)
    out_ref[...] = jnp.dot(s * (1.0 / K), W2_ref[...],
                           preferred_element_type=jnp.float32,
                           precision=lax.Precision.HIGHEST) + b2_ref[...]


_fold_call = pl.pallas_call(
    _fold_body,
    out_shape=[jax.ShapeDtypeStruct((64, 128), jnp.float32),
               jax.ShapeDtypeStruct((1, 128), jnp.float32)],
)

_prep_call = pl.pallas_call(
    _prep_body,
    grid=(NB, K // KG),
    in_specs=[
        pl.BlockSpec((RQ, 3), lambda nb, kg: (nb, 0)),
        pl.BlockSpec((N, 3), lambda nb, kg: (0, 0)),
        pl.BlockSpec((3, RQ), lambda nb, kg: (0, nb)),
        pl.BlockSpec((RQ, D_IN), lambda nb, kg: (nb, 0)),
        pl.BlockSpec((3, 64), lambda nb, kg: (0, 0)),
        pl.BlockSpec((D_IN, 128), lambda nb, kg: (0, 0)),
        pl.BlockSpec((1, 128), lambda nb, kg: (0, 0)),
    ],
    out_specs=[
        pl.BlockSpec((RQ, TW), lambda nb, kg: (nb, 0)),
        pl.BlockSpec((1, KG, RQ), lambda nb, kg: (kg, 0, nb)),
    ],
    out_shape=[jax.ShapeDtypeStruct((N, TW), jnp.float32),
               jax.ShapeDtypeStruct((K // KG, KG, N), jnp.int32)],
    scratch_shapes=[pltpu.VMEM((N, RQ), jnp.float32)],
)

_agg_call = pl.pallas_call(
    _agg_body,
    grid=(NAB,),
    in_specs=[
        pl.BlockSpec((K, AQ, TW), lambda nab: (0, nab, 0)),
        pl.BlockSpec((AQ, TW), lambda nab: (nab, 0)),
        pl.BlockSpec((64, 128), lambda nab: (0, 0)),
        pl.BlockSpec((1, 64), lambda nab: (0, 0)),
        pl.BlockSpec((128, 128), lambda nab: (0, 0)),
        pl.BlockSpec((1, 128), lambda nab: (0, 0)),
    ],
    out_specs=pl.BlockSpec((AQ, D_OUT), lambda nab: (nab, 0)),
    out_shape=jax.ShapeDtypeStruct((N, D_OUT), jnp.float32),
)


@functools.cache
def _sc_gather_call():
    return functools.partial(
        pl.kernel,
        out_type=jax.ShapeDtypeStruct((_TOTB, TW), jnp.float32),
        mesh=plsc.VectorSubcoreMesh(core_axis_name="c",
                                    subcore_axis_name="s"),
        scratch_types=[pltpu.VMEM((2, _CH), jnp.int32),
                       pltpu.VMEM((2, _CH, TW), jnp.float32),
                       pltpu.SemaphoreType.DMA,
                       pltpu.SemaphoreType.DMA],
    )(_sc_gather_body)


def kernel(points, features, density, pos_W1, pos_b1, pos_W2, pos_b2,
           mlp_W1, mlp_b1, mlp_W2, mlp_b2,
           dw_W1, dw_b1, dw_W2, dw_b2, dw_W3, dw_b3):
    del density, dw_W1, dw_b1, dw_W2, dw_b2, dw_W3, dw_b3
    pT = points.transpose(0, 2, 1)
    W1_top = mlp_W1[:D_IN]
    W1_bot = mlp_W1[D_IN:]
    Wf, cb = _fold_call(pos_W2, W1_bot, mlp_b1.reshape(1, 128),
                        pos_b2.reshape(1, 64))
    scg = _sc_gather_call()
    pb1 = pos_b1.reshape(1, 64)
    b2 = mlp_b2.reshape(1, 128)
    outs = []
    for b in range(B):
        ttab, idx = _prep_call(points[b], points[b], pT[b], features[b],
                               pos_W1, W1_top, cb)
        gt = scg(idx.reshape(_TOTB), ttab)
        outs.append(_agg_call(gt.reshape(K, N, TW), ttab, Wf, pb1,
                              mlp_W2, b2))
    return jnp.stack(outs)


# speedup vs baseline: 13.9172x; 1.1618x over previous
import functools

import jax
import jax.numpy as jnp
from jax import lax
from jax.experimental import pallas as pl
from jax.experimental.pallas import tpu as pltpu
from jax.experimental.pallas import tpu_sc as plsc

B, N, K = 4, 2048, 32
D_IN, D_OUT = 128, 128
RQ = 256
NB = N // RQ
KG = 8
AQ = 128
NAB = N // AQ
TW = 256
_BIG = 3.0e38

_NC, _NS = 2, 16
_NW = _NC * _NS
_TOTB = N * K
_RPW = _TOTB // _NW
_CH = 128
_NCH = _RPW // _CH


def _fold_body(posW2_ref, W1bot_ref, b1_ref, posb2_ref, Wf_ref, cb_ref):
    W1bot = W1bot_ref[...]
    Wf_ref[...] = jnp.dot(posW2_ref[...], W1bot,
                          preferred_element_type=jnp.float32,
                          precision=lax.Precision.HIGHEST)
    cb_ref[...] = b1_ref[...] + jnp.dot(posb2_ref[...], W1bot,
                                        preferred_element_type=jnp.float32,
                                        precision=lax.Precision.HIGHEST)


def _prep_body(pq_ref, pc_ref, ptq_ref, f_ref, posW1_ref, W1top_ref, cb_ref,
               ttab_ref, idx_ref, vals_ref, m_ref):
    kg = pl.program_id(1)

    @pl.when(kg == 0)
    def _init():
        ttab_ref[:, 0:64] = jnp.dot(pq_ref[...], posW1_ref[...],
                                    preferred_element_type=jnp.float32,
                                    precision=lax.Precision.HIGHEST)
        ttab_ref[:, 64:128] = jnp.zeros((RQ, 64), jnp.float32)
        ttab_ref[:, 128:256] = jnp.dot(
            f_ref[...], W1top_ref[...],
            preferred_element_type=jnp.float32,
            precision=lax.Precision.HIGHEST) + cb_ref[...]
        pc = pc_ref[...]
        pn = jnp.sum(pc * pc, axis=1, keepdims=True)
        dots = jnp.dot(pc, ptq_ref[...],
                       preferred_element_type=jnp.float32,
                       precision=lax.Precision.HIGHEST)
        vals0 = pn - 2.0 * dots
        vals_ref[...] = vals0
        m_ref[...] = jnp.min(vals0, axis=0, keepdims=True)

    vals = vals_ref[...]
    subf = lax.broadcasted_iota(jnp.int32, (N, RQ), 0).astype(jnp.float32)
    m = m_ref[...]
    for j in range(KG):
        am = jnp.min(jnp.where(vals == m, subf, _BIG),
                     axis=0, keepdims=True)
        idx_ref[0, j] = am.reshape(RQ).astype(jnp.int32)
        m = jnp.min(jnp.where(vals > m, vals, _BIG),
                    axis=0, keepdims=True)
    m_ref[...] = m


def _sc_gather_body(idx_hbm, ttab_hbm, out_hbm, idx_v, rows_v, sem0, sem1):
    wid = lax.axis_index("s") * _NC + lax.axis_index("c")
    base = wid * _RPW
    sems = (sem0, sem1)

    for bsel in range(2):
        off0 = pl.multiple_of(base + bsel * _CH, _CH)
        pltpu.sync_copy(idx_hbm.at[pl.ds(off0, _CH)], idx_v.at[bsel])
        pltpu.async_copy(ttab_hbm.at[idx_v.at[bsel]], rows_v.at[bsel],
                         sems[bsel])

    def pair(g, carry):
        for bsel in range(2):
            c = g * 2 + bsel
            off = pl.multiple_of(base + c * _CH, _CH)
            pltpu.make_async_copy(ttab_hbm.at[idx_v.at[bsel]],
                                  rows_v.at[bsel], sems[bsel]).wait()
            pltpu.sync_copy(rows_v.at[bsel], out_hbm.at[pl.ds(off, _CH)])

            @pl.when(c + 2 < _NCH)
            def _next():
                offn = pl.multiple_of(base + (c + 2) * _CH, _CH)
                pltpu.sync_copy(idx_hbm.at[pl.ds(offn, _CH)], idx_v.at[bsel])
                pltpu.async_copy(ttab_hbm.at[idx_v.at[bsel]],
                                 rows_v.at[bsel], sems[bsel])
        return carry

    lax.fori_loop(0, _NCH // 2, pair, 0)


def _agg_body(gt_ref, ttab_ref, Wf_ref, pb1_ref, W2_ref, b2_ref, out_ref):
    gt = gt_ref[...]
    qc = ttab_ref[:, 0:64]
    a = jnp.maximum(gt[:, :, 0:64] - qc[None] + pb1_ref[...][None], 0.0)
    mm = jnp.dot(a.reshape(K * AQ, 64), Wf_ref[...],
                 preferred_element_type=jnp.float32,
                 precision=lax.Precision.HIGHEST)
    z = gt[:, :, 128:256] + mm.reshape(K, AQ, 128)
    s = jnp.sum(jnp.maximum(z, 0.0), axis=0)
    out_ref[...] = jnp.dot(s * (1.0 / K), W2_ref[...],
                           preferred_element_type=jnp.float32,
                           precision=lax.Precision.HIGHEST) + b2_ref[...]


_fold_call = pl.pallas_call(
    _fold_body,
    out_shape=[jax.ShapeDtypeStruct((64, 128), jnp.float32),
               jax.ShapeDtypeStruct((1, 128), jnp.float32)],
)

_prep_call = pl.pallas_call(
    _prep_body,
    grid=(NB, K // KG),
    in_specs=[
        pl.BlockSpec((RQ, 3), lambda nb, kg: (nb, 0)),
        pl.BlockSpec((N, 3), lambda nb, kg: (0, 0)),
        pl.BlockSpec((3, RQ), lambda nb, kg: (0, nb)),
        pl.BlockSpec((RQ, D_IN), lambda nb, kg: (nb, 0)),
        pl.BlockSpec((3, 64), lambda nb, kg: (0, 0)),
        pl.BlockSpec((D_IN, 128), lambda nb, kg: (0, 0)),
        pl.BlockSpec((1, 128), lambda nb, kg: (0, 0)),
    ],
    out_specs=[
        pl.BlockSpec((RQ, TW), lambda nb, kg: (nb, 0)),
        pl.BlockSpec((1, KG, RQ), lambda nb, kg: (kg, 0, nb)),
    ],
    out_shape=[jax.ShapeDtypeStruct((N, TW), jnp.float32),
               jax.ShapeDtypeStruct((K // KG, KG, N), jnp.int32)],
    scratch_shapes=[pltpu.VMEM((N, RQ), jnp.float32),
                    pltpu.VMEM((1, RQ), jnp.float32)],
)

_agg_call = pl.pallas_call(
    _agg_body,
    grid=(NAB,),
    in_specs=[
        pl.BlockSpec((K, AQ, TW), lambda nab: (0, nab, 0)),
        pl.BlockSpec((AQ, TW), lambda nab: (nab, 0)),
        pl.BlockSpec((64, 128), lambda nab: (0, 0)),
        pl.BlockSpec((1, 64), lambda nab: (0, 0)),
        pl.BlockSpec((128, 128), lambda nab: (0, 0)),
        pl.BlockSpec((1, 128), lambda nab: (0, 0)),
    ],
    out_specs=pl.BlockSpec((AQ, D_OUT), lambda nab: (nab, 0)),
    out_shape=jax.ShapeDtypeStruct((N, D_OUT), jnp.float32),
)


@functools.cache
def _sc_gather_call():
    return functools.partial(
        pl.kernel,
        out_type=jax.ShapeDtypeStruct((_TOTB, TW), jnp.float32),
        mesh=plsc.VectorSubcoreMesh(core_axis_name="c",
                                    subcore_axis_name="s"),
        scratch_types=[pltpu.VMEM((2, _CH), jnp.int32),
                       pltpu.VMEM((2, _CH, TW), jnp.float32),
                       pltpu.SemaphoreType.DMA,
                       pltpu.SemaphoreType.DMA],
    )(_sc_gather_body)


def kernel(points, features, density, pos_W1, pos_b1, pos_W2, pos_b2,
           mlp_W1, mlp_b1, mlp_W2, mlp_b2,
           dw_W1, dw_b1, dw_W2, dw_b2, dw_W3, dw_b3):
    del density, dw_W1, dw_b1, dw_W2, dw_b2, dw_W3, dw_b3
    pT = points.transpose(0, 2, 1)
    W1_top = mlp_W1[:D_IN]
    W1_bot = mlp_W1[D_IN:]
    Wf, cb = _fold_call(pos_W2, W1_bot, mlp_b1.reshape(1, 128),
                        pos_b2.reshape(1, 64))
    scg = _sc_gather_call()
    pb1 = pos_b1.reshape(1, 64)
    b2 = mlp_b2.reshape(1, 128)
    outs = []
    for b in range(B):
        ttab, idx = _prep_call(points[b], points[b], pT[b], features[b],
                               pos_W1, W1_top, cb)
        gt = scg(idx.reshape(_TOTB), ttab)
        outs.append(_agg_call(gt.reshape(K, N, TW), ttab, Wf, pb1,
                              mlp_W2, b2))
    return jnp.stack(outs)
